# baseline (speedup 1.0000x reference)
import functools

import jax
import jax.numpy as jnp
from jax import lax
from jax.experimental import pallas as pl
from jax.experimental.pallas import tpu as pltpu
from jax.experimental.pallas import tpu_sc as plsc

DIM = 16
NW = 32
CHUNK = 800
NBUF = 4


def _tc_precompute(x2d, ba, bb, b1t):
    n_rows = x2d.shape[0]
    blk = n_rows
    grid = (1,)

    def body(x_ref, ba_ref, bb_ref, b_ref, u_ref, v_ref):
        xb = x_ref[...]
        u_ref[...] = (
            jnp.dot(xb, ba_ref[...], preferred_element_type=jnp.float32)
            + b_ref[...]
        )
        v_ref[...] = jnp.dot(xb, bb_ref[...], preferred_element_type=jnp.float32)

    return pl.pallas_call(
        body,
        grid=grid,
        in_specs=[
            pl.BlockSpec((blk, 8 * DIM), lambda i: (i, 0)),
            pl.BlockSpec((8 * DIM, 8 * DIM), lambda i: (0, 0)),
            pl.BlockSpec((8 * DIM, 8 * DIM), lambda i: (0, 0)),
            pl.BlockSpec((1, 8 * DIM), lambda i: (0, 0)),
        ],
        out_specs=[
            pl.BlockSpec((blk, 8 * DIM), lambda i: (i, 0)),
            pl.BlockSpec((blk, 8 * DIM), lambda i: (i, 0)),
        ],
        out_shape=[
            jax.ShapeDtypeStruct((n_rows, 8 * DIM), jnp.float32),
            jax.ShapeDtypeStruct((n_rows, 8 * DIM), jnp.float32),
        ],
    )(x2d, ba, bb, b1t)


def _make_sc_kernel(n_edges):
    per_w = n_edges // NW
    chunk = CHUNK
    nbuf = NBUF
    n_chunks = per_w // chunk
    n_rounds = n_chunks // nbuf
    groups = chunk // DIM
    nc = 2
    ns = 16
    mesh = plsc.VectorSubcoreMesh(
        core_axis_name="c", subcore_axis_name="s", num_cores=nc, num_subcores=ns
    )

    @functools.partial(
        pl.kernel,
        mesh=mesh,
        out_type=jax.ShapeDtypeStruct((n_edges,), jnp.float32),
        scratch_types=[
            pltpu.VMEM((nbuf, chunk), jnp.int32),
            pltpu.VMEM((nbuf, chunk), jnp.int32),
            pltpu.VMEM((nbuf * chunk, DIM), jnp.float32),
            pltpu.VMEM((nbuf * chunk, DIM), jnp.float32),
            pltpu.VMEM((nbuf, chunk), jnp.float32),
            pltpu.VMEM((DIM, DIM), jnp.float32),
            pltpu.VMEM((DIM,), jnp.float32),
        ] + [pltpu.SemaphoreType.DMA] * (3 * nbuf),
        compiler_params=pltpu.CompilerParams(
            needs_layout_passes=False, use_tc_tiling_on_sc=False
        ),
    )
    def sc_kernel(u_hbm, v_hbm, el_hbm, w2s_hbm, b2s_hbm, out_hbm,
                  idxh_v, idxt_v, ubuf, vbuf, outbuf, w2v, b2v, *sems):
        wid = lax.axis_index("s") * nc + lax.axis_index("c")
        base0 = wid * per_w
        si = sems[0:nbuf]
        sg = sems[nbuf:2 * nbuf]
        so = sems[2 * nbuf:3 * nbuf]
        pltpu.sync_copy(w2s_hbm, w2v)
        pltpu.sync_copy(b2s_hbm, b2v)
        w2diags = [w2v[d] for d in range(DIM)]
        b2vec = b2v[...]
        iota16 = lax.iota(jnp.int32, DIM)
        colvs = [jnp.bitwise_and(iota16 + d, DIM - 1) for d in range(DIM)]

        def fire_idx(ci, b):
            base = base0 + ci * chunk
            pltpu.async_copy(el_hbm.at[0, pl.ds(base, chunk)], idxh_v.at[b], si[b])
            pltpu.async_copy(el_hbm.at[1, pl.ds(base, chunk)], idxt_v.at[b], si[b])

        def fire_gather(b):
            pltpu.make_async_copy(
                el_hbm.at[0, pl.ds(0, chunk)], idxh_v.at[b], si[b]).wait()
            pltpu.make_async_copy(
                el_hbm.at[1, pl.ds(0, chunk)], idxt_v.at[b], si[b]).wait()
            pltpu.async_copy(
                u_hbm.at[idxh_v.at[b]], ubuf.at[pl.ds(b * chunk, chunk)], sg[b])
            pltpu.async_copy(
                v_hbm.at[idxt_v.at[b]], vbuf.at[pl.ds(b * chunk, chunk)], sg[b])

        def drain_gather(b):
            pltpu.make_async_copy(
                u_hbm.at[idxh_v.at[b]],
                ubuf.at[pl.ds(b * chunk, chunk)], sg[b]).wait()
            pltpu.make_async_copy(
                v_hbm.at[idxt_v.at[b]],
                vbuf.at[pl.ds(b * chunk, chunk)], sg[b]).wait()

        def drain_out(b):
            pltpu.make_async_copy(
                outbuf.at[b], out_hbm.at[pl.ds(0, chunk)], so[b]).wait()

        def compute(ci, b):
            rbase = b * chunk

            def grp_body(g, c2):
                rows = rbase + g * DIM + iota16
                acc = b2vec
                for d in range(DIM):
                    uu = plsc.load_gather(ubuf, [rows, colvs[d]])
                    vv = plsc.load_gather(vbuf, [rows, colvs[d]])
                    acc = acc + jnp.maximum(uu + vv, 0.0) * w2diags[d]
                outbuf[b, pl.ds(g * DIM, DIM)] = 1.0 / (1.0 + jnp.exp(-acc))
                return c2

            lax.fori_loop(0, groups, grp_body, 0)
            pltpu.async_copy(
                outbuf.at[b],
                out_hbm.at[pl.ds(base0 + ci * chunk, chunk)], so[b])

        def maybe(cond, fn):
            if isinstance(cond, bool):
                if cond:
                    fn()
            else:
                pl.when(cond)(fn)

        def phase(ci, b):
            drain_gather(b)
            maybe(ci + nbuf < n_chunks, lambda: fire_idx(ci + nbuf, b))
            maybe(ci >= nbuf, lambda: drain_out(b))
            maybe(ci + nbuf - 1 < n_chunks,
                  lambda: fire_gather((b + nbuf - 1) % nbuf))
            compute(ci, b)

        for k in range(nbuf):
            fire_idx(k, k)
        for k in range(nbuf - 1):
            fire_gather(k)

        def round_body(t, carry):
            for k in range(nbuf):
                phase(nbuf * t + k, k)
            return carry

        lax.fori_loop(0, n_rounds, round_body, 0)
        for ci in range(nbuf * n_rounds, n_chunks):
            phase(ci, ci % nbuf)
        for tb in range(nbuf):
            drain_out((n_chunks - nbuf + tb) % nbuf)

    return sc_kernel


def kernel(x, edge_label_index, W1, b1, W2, b2):
    n_nodes = x.shape[0]
    n_edges = edge_label_index.shape[1]
    el = edge_label_index.astype(jnp.int32)
    x2d = x.reshape(n_nodes // 8, 8 * DIM)
    eye8 = jnp.eye(8, dtype=jnp.float32)
    ba = jnp.kron(eye8, W1[:DIM, :])
    bb = jnp.kron(eye8, W1[DIM:, :])
    b1t = jnp.tile(b1, 8).reshape(1, 8 * DIM)
    U2d, V2d = _tc_precompute(x2d, ba, bb, b1t)
    U = U2d.reshape(n_nodes, DIM)
    V = V2d.reshape(n_nodes, DIM)
    j = jnp.arange(DIM)
    w2s = W2.reshape(DIM)[(j[None, :] + j[:, None]) % DIM]
    b2s = jnp.broadcast_to(b2.reshape(1), (DIM,))
    out = _make_sc_kernel(n_edges)(U, V, el, w2s, b2s)
    return out.reshape(n_edges, 1)


# speedup vs baseline: 1.0074x; 1.0074x over previous
import functools

import jax
import jax.numpy as jnp
from jax import lax
from jax.experimental import pallas as pl
from jax.experimental.pallas import tpu as pltpu
from jax.experimental.pallas import tpu_sc as plsc

DIM = 16
NW = 32
CHUNK = 800
NBUF = 3


def _tc_precompute(x2d, ba, bb, b1t):
    n_rows = x2d.shape[0]
    blk = n_rows
    grid = (1,)

    def body(x_ref, ba_ref, bb_ref, b_ref, u_ref, v_ref):
        xb = x_ref[...]
        u_ref[...] = (
            jnp.dot(xb, ba_ref[...], preferred_element_type=jnp.float32)
            + b_ref[...]
        )
        v_ref[...] = jnp.dot(xb, bb_ref[...], preferred_element_type=jnp.float32)

    return pl.pallas_call(
        body,
        grid=grid,
        in_specs=[
            pl.BlockSpec((blk, 8 * DIM), lambda i: (i, 0)),
            pl.BlockSpec((8 * DIM, 8 * DIM), lambda i: (0, 0)),
            pl.BlockSpec((8 * DIM, 8 * DIM), lambda i: (0, 0)),
            pl.BlockSpec((1, 8 * DIM), lambda i: (0, 0)),
        ],
        out_specs=[
            pl.BlockSpec((blk, 8 * DIM), lambda i: (i, 0)),
            pl.BlockSpec((blk, 8 * DIM), lambda i: (i, 0)),
        ],
        out_shape=[
            jax.ShapeDtypeStruct((n_rows, 8 * DIM), jnp.float32),
            jax.ShapeDtypeStruct((n_rows, 8 * DIM), jnp.float32),
        ],
    )(x2d, ba, bb, b1t)


def _make_sc_kernel(n_edges):
    per_w = n_edges // NW
    chunk = CHUNK
    nbuf = NBUF
    n_chunks = per_w // chunk
    n_rounds = n_chunks // nbuf
    groups = chunk // DIM
    nc = 2
    ns = 16
    mesh = plsc.VectorSubcoreMesh(
        core_axis_name="c", subcore_axis_name="s", num_cores=nc, num_subcores=ns
    )

    @functools.partial(
        pl.kernel,
        mesh=mesh,
        out_type=jax.ShapeDtypeStruct((n_edges,), jnp.float32),
        scratch_types=[
            pltpu.VMEM((nbuf, chunk), jnp.int32),
            pltpu.VMEM((nbuf, chunk), jnp.int32),
            pltpu.VMEM((nbuf * chunk, DIM), jnp.float32),
            pltpu.VMEM((nbuf * chunk, DIM), jnp.float32),
            pltpu.VMEM((nbuf, chunk), jnp.float32),
            pltpu.VMEM((DIM, DIM), jnp.float32),
            pltpu.VMEM((DIM,), jnp.float32),
        ] + [pltpu.SemaphoreType.DMA] * (3 * nbuf),
        compiler_params=pltpu.CompilerParams(
            needs_layout_passes=False, use_tc_tiling_on_sc=False
        ),
    )
    def sc_kernel(u_hbm, v_hbm, el_hbm, w2s_hbm, b2s_hbm, out_hbm,
                  idxh_v, idxt_v, ubuf, vbuf, outbuf, w2v, b2v, *sems):
        wid = lax.axis_index("s") * nc + lax.axis_index("c")
        base0 = wid * per_w
        si = sems[0:nbuf]
        sg = sems[nbuf:2 * nbuf]
        so = sems[2 * nbuf:3 * nbuf]
        pltpu.sync_copy(w2s_hbm, w2v)
        pltpu.sync_copy(b2s_hbm, b2v)
        w2diags = [w2v[d] for d in range(DIM)]
        b2vec = b2v[...]
        iota16 = lax.iota(jnp.int32, DIM)
        colvs = [jnp.bitwise_and(iota16 + d, DIM - 1) for d in range(DIM)]

        def fire_idx(ci, b):
            base = base0 + ci * chunk
            pltpu.async_copy(el_hbm.at[0, pl.ds(base, chunk)], idxh_v.at[b], si[b])
            pltpu.async_copy(el_hbm.at[1, pl.ds(base, chunk)], idxt_v.at[b], si[b])

        def fire_gather(b):
            pltpu.make_async_copy(
                el_hbm.at[0, pl.ds(0, chunk)], idxh_v.at[b], si[b]).wait()
            pltpu.make_async_copy(
                el_hbm.at[1, pl.ds(0, chunk)], idxt_v.at[b], si[b]).wait()
            pltpu.async_copy(
                u_hbm.at[idxh_v.at[b]], ubuf.at[pl.ds(b * chunk, chunk)], sg[b])
            pltpu.async_copy(
                v_hbm.at[idxt_v.at[b]], vbuf.at[pl.ds(b * chunk, chunk)], sg[b])

        def drain_gather(b):
            pltpu.make_async_copy(
                u_hbm.at[idxh_v.at[b]],
                ubuf.at[pl.ds(b * chunk, chunk)], sg[b]).wait()
            pltpu.make_async_copy(
                v_hbm.at[idxt_v.at[b]],
                vbuf.at[pl.ds(b * chunk, chunk)], sg[b]).wait()

        def drain_out(b):
            pltpu.make_async_copy(
                outbuf.at[b], out_hbm.at[pl.ds(0, chunk)], so[b]).wait()

        def compute(ci, b):
            rbase = b * chunk

            def grp_body(g, c2):
                rows = rbase + g * DIM + iota16
                acc = b2vec
                for d in range(DIM):
                    uu = plsc.load_gather(ubuf, [rows, colvs[d]])
                    vv = plsc.load_gather(vbuf, [rows, colvs[d]])
                    acc = acc + jnp.maximum(uu + vv, 0.0) * w2diags[d]
                outbuf[b, pl.ds(g * DIM, DIM)] = 1.0 / (1.0 + jnp.exp(-acc))
                return c2

            lax.fori_loop(0, groups, grp_body, 0)
            pltpu.async_copy(
                outbuf.at[b],
                out_hbm.at[pl.ds(base0 + ci * chunk, chunk)], so[b])

        def maybe(cond, fn):
            if isinstance(cond, bool):
                if cond:
                    fn()
            else:
                pl.when(cond)(fn)

        def phase(ci, b):
            drain_gather(b)
            maybe(ci + nbuf < n_chunks, lambda: fire_idx(ci + nbuf, b))
            maybe(ci >= nbuf, lambda: drain_out(b))
            maybe(ci + nbuf - 1 < n_chunks,
                  lambda: fire_gather((b + nbuf - 1) % nbuf))
            compute(ci, b)

        for k in range(nbuf):
            fire_idx(k, k)
        for k in range(nbuf - 1):
            fire_gather(k)

        def round_body(t, carry):
            for k in range(nbuf):
                phase(nbuf * t + k, k)
            return carry

        lax.fori_loop(0, n_rounds, round_body, 0)
        for ci in range(nbuf * n_rounds, n_chunks):
            phase(ci, ci % nbuf)
        for tb in range(nbuf):
            drain_out((n_chunks - nbuf + tb) % nbuf)

    return sc_kernel


def kernel(x, edge_label_index, W1, b1, W2, b2):
    n_nodes = x.shape[0]
    n_edges = edge_label_index.shape[1]
    el = edge_label_index.astype(jnp.int32)
    x2d = x.reshape(n_nodes // 8, 8 * DIM)
    eye8 = jnp.eye(8, dtype=jnp.float32)
    ba = jnp.kron(eye8, W1[:DIM, :])
    bb = jnp.kron(eye8, W1[DIM:, :])
    b1t = jnp.tile(b1, 8).reshape(1, 8 * DIM)
    U2d, V2d = _tc_precompute(x2d, ba, bb, b1t)
    U = U2d.reshape(n_nodes, DIM)
    V = V2d.reshape(n_nodes, DIM)
    j = jnp.arange(DIM)
    w2s = W2.reshape(DIM)[(j[None, :] + j[:, None]) % DIM]
    b2s = jnp.broadcast_to(b2.reshape(1), (DIM,))
    out = _make_sc_kernel(n_edges)(U, V, el, w2s, b2s)
    return out.reshape(n_edges, 1)
